# baseline (speedup 1.0000x reference)
import functools

import jax
import jax.numpy as jnp
from jax import lax
from jax.experimental import pallas as pl
from jax.experimental.pallas import tpu as pltpu
from jax.experimental.pallas import tpu_sc as plsc

D_MODEL = 768
B_TOTAL = 4 * 4096
NUM_WORKERS = 32
B_PER_W = B_TOTAL // NUM_WORKERS
CHUNK = 32
NCHUNK = B_PER_W // CHUNK
NBUF = 3
SBUF = 2
NS = 16

_mesh = plsc.VectorSubcoreMesh(core_axis_name="c", subcore_axis_name="s")


@functools.partial(
    pl.kernel,
    mesh=_mesh,
    out_type=jax.ShapeDtypeStruct((B_TOTAL, D_MODEL), jnp.float32),
    scratch_types=[
        pltpu.VMEM((NCHUNK, CHUNK), jnp.int32),
        pltpu.VMEM((NBUF, CHUNK, D_MODEL), jnp.float32),
        pltpu.VMEM_SHARED((NS, SBUF, CHUNK, D_MODEL), jnp.float32),
        pltpu.SemaphoreType.DMA((NBUF,)),
        pltpu.SemaphoreType.DMA((NBUF,)),
        pltpu.SemaphoreType.DMA((SBUF,)),
    ],
)
def _gather_kernel(idx_hbm, table_hbm, out_hbm, idx_v, bufs, shared,
                   gsem, csem, ssem):
    sid = lax.axis_index("s")
    wid = sid * 2 + lax.axis_index("c")
    base = wid * B_PER_W
    pltpu.sync_copy(idx_hbm.at[pl.ds(wid * NCHUNK, NCHUNK)], idx_v)

    a_h = [None] * NBUF
    b_h = [None] * NBUF
    c_h = [None] * SBUF

    def stage_a(i):
        b = i % NBUF
        a_h[b] = pltpu.async_copy(
            table_hbm.at[idx_v.at[i]], bufs.at[b], gsem.at[b])

    def stage_b(i):
        b = i % NBUF
        s = i % SBUF
        a_h[b].wait()
        if c_h[s] is not None:
            c_h[s].wait()
        b_h[b] = pltpu.async_copy(
            bufs.at[b], shared.at[sid].at[s], csem.at[b])

    def stage_c(i):
        b = i % NBUF
        s = i % SBUF
        b_h[b].wait()
        c_h[s] = pltpu.async_copy(
            shared.at[sid].at[s],
            out_hbm.at[pl.ds(base + i * CHUNK, CHUNK)],
            ssem.at[s])

    for step in range(NCHUNK + 2):
        if step < NCHUNK:
            stage_a(step)
        if step >= 1 and step - 1 < NCHUNK:
            stage_b(step - 1)
        if step >= 2:
            stage_c(step - 2)
    for s in range(SBUF):
        if c_h[s] is not None:
            c_h[s].wait()


def kernel(input_ids, table):
    ids = input_ids.reshape(-1, CHUNK).astype(jnp.int32)
    out = _gather_kernel(ids, table)
    return out.reshape(input_ids.shape + (D_MODEL,))
